# baseline (speedup 1.0000x reference)
import functools
import math

import jax
import jax.numpy as jnp
from jax import lax
from jax.experimental import pallas as pl
from jax.experimental.pallas import tpu as pltpu

BLK = 128


def _corr_kernel(qh_ref, ql_ref, kh_ref, kl_ref, d_ref, p_ref, *,
                 nblk, apn, jpn, ab, jb):
    ap = pl.program_id(1)
    jp = pl.program_id(2)

    @pl.when(jnp.logical_and(ap == 0, jp == 0))
    def _init():
        p_ref[...] = jnp.zeros_like(p_ref)

    qh = qh_ref[0]
    ql = ql_ref[0]
    kh = kh_ref[0]
    kl = kl_ref[0]
    dn = (((1,), (0,)), ((), ()))
    s = lax.dot_general(qh, kh, dn, preferred_element_type=jnp.float32) + (
        lax.dot_general(qh, kl, dn, preferred_element_type=jnp.float32)
        + lax.dot_general(ql, kh, dn, preferred_element_type=jnp.float32))

    for ai in range(ab):
        for ji in range(jb):
            c = jnp.mod(jp * jb + ji - ap * ab - ai, nblk)
            blk = s[ai * BLK:(ai + 1) * BLK, ji * BLK:(ji + 1) * BLK]
            p_ref[pl.ds(c * BLK, BLK), :] += blk

    @pl.when(jnp.logical_and(ap == apn - 1, jp == jpn - 1))
    def _finale():
        rows = lax.broadcasted_iota(jnp.int32, (BLK, 2 * BLK), 0)
        for cb in range(nblk):
            a = p_ref[cb * BLK:(cb + 1) * BLK, :]
            nxt = ((cb + 1) % nblk) * BLK
            b2 = p_ref[nxt:nxt + BLK, :]
            r = jnp.concatenate([a, b2], axis=1)
            for bit in range(7):
                sh = 1 << bit
                mask = ((rows >> bit) & 1) == 1
                r = jnp.where(mask, jnp.roll(r, -sh, axis=1), r)
            colsum = jnp.sum(r, axis=0, keepdims=True)
            d_ref[0, :, cb * BLK:(cb + 1) * BLK] = colsum[:, :BLK]


def _corr(qh, ql, kh, kl, interpret=False):
    B, L, C = qh.shape
    nblk = L // BLK
    ab, jb = 8, 4
    apn, jpn = nblk // ab, nblk // jb
    qspec = pl.BlockSpec((1, ab * BLK, C), lambda b, ap, jp: (b, ap, 0))
    kspec = pl.BlockSpec((1, C, jb * BLK), lambda b, ap, jp: (b, 0, jp))
    return pl.pallas_call(
        functools.partial(_corr_kernel, nblk=nblk, apn=apn, jpn=jpn,
                          ab=ab, jb=jb),
        grid=(B, apn, jpn),
        in_specs=[qspec, qspec, kspec, kspec],
        out_specs=pl.BlockSpec((1, 1, L), lambda b, ap, jp: (b, 0, 0)),
        out_shape=jax.ShapeDtypeStruct((B, 1, L), jnp.float32),
        scratch_shapes=[pltpu.VMEM((L, BLK), jnp.float32)],
        compiler_params=pltpu.CompilerParams(
            dimension_semantics=("parallel", "arbitrary", "arbitrary")),
        interpret=interpret,
    )(qh, ql, kh, kl)


def _agg_kernel(d_ref, v_ref, o_ref, idx_ref, al_ref, *,
                B, L, C, topk):
    b = pl.program_id(0)
    cc = pl.program_id(1)

    @pl.when(cc == 0)
    def _select():
        ii = lax.broadcasted_iota(jnp.int32, (1, L), 1)
        m = (d_ref[0:1, :] + d_ref[1:2, :]) * 0.5
        msel = m
        for i in range(topk):
            mx = jnp.max(msel)
            pos = jnp.min(jnp.where(msel == mx, ii, L))
            idx_ref[i] = pos
            msel = jnp.where(ii == pos, -jnp.inf, msel)
        sel = msel == -jnp.inf
        x = d_ref[...] * (1.0 / C)
        xm = jnp.where(sel, x, -jnp.inf)
        mxb = jnp.max(xm, axis=1, keepdims=True)
        e = jnp.where(sel, jnp.exp(x - mxb), 0.0)
        af = e / jnp.sum(e, axis=1, keepdims=True)
        for bb in range(B):
            for i in range(topk):
                pos = idx_ref[i]
                al_ref[bb, i] = jnp.sum(
                    jnp.where(ii == pos, af[bb:bb + 1, :], 0.0))

    vb = v_ref[0]
    acc = al_ref[b, 0] * pltpu.roll(vb, L - idx_ref[0], 0)
    for i in range(1, topk):
        acc = acc + al_ref[b, i] * pltpu.roll(vb, L - idx_ref[i], 0)
    o_ref[0] = acc


_CW = 128


def _agg(d, v2, topk, interpret=False):
    B, L, C = v2.shape
    vspec = pl.BlockSpec((1, L, _CW), lambda b, cc: (b, 0, cc))
    return pl.pallas_call(
        functools.partial(_agg_kernel, B=B, L=L, C=C, topk=topk),
        grid=(B, C // _CW),
        in_specs=[pl.BlockSpec((B, L), lambda b, cc: (0, 0)), vspec],
        out_specs=vspec,
        out_shape=jax.ShapeDtypeStruct((B, L, C), jnp.float32),
        scratch_shapes=[
            pltpu.SMEM((topk,), jnp.int32),
            pltpu.SMEM((B, topk), jnp.float32),
        ],
        compiler_params=pltpu.CompilerParams(
            dimension_semantics=("parallel", "arbitrary")),
        interpret=interpret,
    )(d, v2)


def kernel(q, k, v, interpret=False):
    B, L, H, D = q.shape
    C = H * D
    q2 = q.reshape(B, L, C)
    k2 = k.reshape(B, L, C)
    kh = k2.astype(jnp.bfloat16)
    kl = (k2 - kh.astype(jnp.float32)).astype(jnp.bfloat16)
    qt = jnp.swapaxes(q2, 1, 2)
    qh = qt.astype(jnp.bfloat16)
    ql = (qt - qh.astype(jnp.float32)).astype(jnp.bfloat16)
    d = _corr(kh, kl, qh, ql, interpret=interpret).reshape(B, L)
    topk = int(math.log(L))
    out = _agg(d, v.reshape(B, L, C), topk, interpret=interpret)
    return out.reshape(B, L, H, D)


# speedup vs baseline: 1.4124x; 1.4124x over previous
import functools
import math

import jax
import jax.numpy as jnp
from jax import lax
from jax.experimental import pallas as pl
from jax.experimental.pallas import tpu as pltpu

BLK = 128


def _corr_kernel(kh_ref, kl_ref, qh_ref, ql_ref, p_ref, acc_ref, *,
                 nblk, apn, jpn, ab, jb):
    ap = pl.program_id(1)
    jp = pl.program_id(2)

    @pl.when(jnp.logical_and(ap == 0, jp == 0))
    def _init():
        acc_ref[...] = jnp.zeros_like(acc_ref)

    kh = kh_ref[0]
    kl = kl_ref[0]
    qh = qh_ref[0]
    ql = ql_ref[0]
    dn = (((1,), (1,)), ((), ()))
    s = lax.dot_general(kh, qh, dn, preferred_element_type=jnp.float32) + (
        lax.dot_general(kh, ql, dn, preferred_element_type=jnp.float32)
        + lax.dot_general(kl, qh, dn, preferred_element_type=jnp.float32))

    for ai in range(ab):
        for ji in range(jb):
            c = jnp.mod(jp * jb + ji - ap * ab - ai, nblk)
            blk = s[ai * BLK:(ai + 1) * BLK, ji * BLK:(ji + 1) * BLK]
            acc_ref[pl.ds(c * BLK, BLK), :] += blk

    @pl.when(jnp.logical_and(ap == apn - 1, jp == jpn - 1))
    def _flush():
        p_ref[0] = acc_ref[...]


def _corr(kh, kl, qh, ql, interpret=False):
    B, L, C = kh.shape
    nblk = L // BLK
    ab, jb = 8, 8
    apn, jpn = nblk // ab, nblk // jb
    kspec = pl.BlockSpec((1, ab * BLK, C), lambda b, ap, jp: (b, ap, 0))
    qspec = pl.BlockSpec((1, jb * BLK, C), lambda b, ap, jp: (b, jp, 0))
    return pl.pallas_call(
        functools.partial(_corr_kernel, nblk=nblk, apn=apn, jpn=jpn,
                          ab=ab, jb=jb),
        grid=(B, apn, jpn),
        in_specs=[kspec, kspec, qspec, qspec],
        out_specs=pl.BlockSpec((1, L, BLK), lambda b, ap, jp: (b, 0, 0)),
        out_shape=jax.ShapeDtypeStruct((B, L, BLK), jnp.float32),
        scratch_shapes=[pltpu.VMEM((L, BLK), jnp.float32)],
        compiler_params=pltpu.CompilerParams(
            dimension_semantics=("parallel", "arbitrary", "arbitrary")),
        interpret=interpret,
    )(kh, kl, qh, ql)


def _select_kernel(p_ref, idx_ref, al_ref, *, B, L, C, nblk, topk):
    rows = lax.broadcasted_iota(jnp.int32, (BLK, 2 * BLK), 0)
    d = []
    for b in range(B):
        cols = []
        for cb in range(nblk):
            a = p_ref[b, cb * BLK:(cb + 1) * BLK, :]
            nxt = ((cb + 1) % nblk) * BLK
            b2 = p_ref[b, nxt:nxt + BLK, :]
            r = jnp.concatenate([a, b2], axis=1)
            for bit in range(7):
                sh = 1 << bit
                mask = ((rows >> bit) & 1) == 1
                r = jnp.where(mask, jnp.roll(r, -sh, axis=1), r)
            cols.append(jnp.sum(r, axis=0, keepdims=True)[:, :BLK])
        d.append(jnp.concatenate(cols, axis=1))
    ii = lax.broadcasted_iota(jnp.int32, (1, L), 1)
    m = d[0]
    for b in range(1, B):
        m = m + d[b]
    msel = m
    for i in range(topk):
        mx = jnp.max(msel)
        pos = jnp.min(jnp.where(msel == mx, ii, L))
        idx_ref[i] = pos
        msel = jnp.where(ii == pos, -jnp.inf, msel)
    sel = msel == -jnp.inf
    for b in range(B):
        x = d[b] * (1.0 / C)
        xm = jnp.where(sel, x, -jnp.inf)
        mxb = jnp.max(xm)
        e = jnp.where(sel, jnp.exp(x - mxb), 0.0)
        af = e / jnp.sum(e)
        for i in range(topk):
            pos = idx_ref[i]
            al_ref[b, i] = jnp.sum(jnp.where(ii == pos, af, 0.0))


def _select(p, C, topk, interpret=False):
    B, L, _ = p.shape
    nblk = L // BLK
    return pl.pallas_call(
        functools.partial(_select_kernel, B=B, L=L, C=C, nblk=nblk,
                          topk=topk),
        in_specs=[pl.BlockSpec((B, L, BLK), lambda: (0, 0, 0))],
        out_specs=[pl.BlockSpec(memory_space=pltpu.SMEM),
                   pl.BlockSpec(memory_space=pltpu.SMEM)],
        out_shape=[jax.ShapeDtypeStruct((topk,), jnp.int32),
                   jax.ShapeDtypeStruct((B, topk), jnp.float32)],
        interpret=interpret,
    )(p)


def _agg_kernel(idx_ref, al_ref, v_ref, o_ref, vv_ref, *, B, L, topk):
    b = pl.program_id(0)
    vb = v_ref[0]
    vv_ref[pl.ds(0, L), :] = vb
    vv_ref[pl.ds(L, L), :] = vb
    vv_ref[pl.ds(2 * L, 8), :] = vb[0:8, :]
    o_ref[0] = jnp.zeros_like(o_ref[0])
    for i in range(topk):
        s = idx_ref[i]
        a8 = pl.multiple_of((s // 8) * 8, 8)
        sl = vv_ref[pl.ds(a8, L + 8), :]
        fine = s - a8
        for c in range(8):
            @pl.when(fine == c)
            def _(sl=sl, c=c, i=i):
                o_ref[0] += al_ref[b, i] * sl[c:c + L, :]


_CW = 128


def _agg(idx, al, v2, topk, interpret=False):
    B, L, C = v2.shape
    vspec = pl.BlockSpec((1, L, _CW), lambda b, cc: (b, 0, cc))
    return pl.pallas_call(
        functools.partial(_agg_kernel, B=B, L=L, topk=topk),
        grid=(B, C // _CW),
        in_specs=[pl.BlockSpec(memory_space=pltpu.SMEM),
                  pl.BlockSpec(memory_space=pltpu.SMEM),
                  vspec],
        out_specs=vspec,
        out_shape=jax.ShapeDtypeStruct((B, L, C), jnp.float32),
        scratch_shapes=[pltpu.VMEM((2 * L + 8, _CW), jnp.float32)],
        compiler_params=pltpu.CompilerParams(
            dimension_semantics=("parallel", "arbitrary")),
        interpret=interpret,
    )(idx, al, v2)


def kernel(q, k, v, interpret=False):
    B, L, H, D = q.shape
    C = H * D
    q2 = q.reshape(B, L, C)
    k2 = k.reshape(B, L, C)
    kh = k2.astype(jnp.bfloat16)
    kl = (k2 - kh.astype(jnp.float32)).astype(jnp.bfloat16)
    qh = q2.astype(jnp.bfloat16)
    ql = (q2 - qh.astype(jnp.float32)).astype(jnp.bfloat16)
    p = _corr(kh, kl, qh, ql, interpret=interpret)
    topk = int(math.log(L))
    idx, al = _select(p, C, topk, interpret=interpret)
    out = _agg(idx, al, v.reshape(B, L, C), topk, interpret=interpret)
    return out.reshape(B, L, H, D)
